# Initial kernel scaffold; baseline (speedup 1.0000x reference)
import jax
import jax.numpy as jnp
from jax.experimental import pallas as pl


def kernel(x, edge_index, edge_attr, a, b, gamma1, gamma2, bias, W1, b1, W2, b2):
    raise NotImplementedError("write your pallas kernel here")


# speedup vs baseline: 1.0576x; 1.0576x over previous
import functools

import jax
import jax.numpy as jnp
from jax import lax
from jax.experimental import pallas as pl
from jax.experimental.pallas import tpu as pltpu

N = 100000
E = 1600000
H = 20
EH = 32
EO = 10

E_BLK = 12800
N_BLK = 1000


def _eac_body(attr_ref, w1_ref, b1_ref, w2_ref, b2_ref, out_ref):
    d = attr_ref[:, 0]
    v = d / jnp.float32(0.1)
    bin_idx = jnp.clip(v.astype(jnp.int32), 0, 9)
    cols = lax.broadcasted_iota(jnp.int32, (d.shape[0], 10), 1)
    one_hot = (cols == bin_idx[:, None]).astype(jnp.float32)
    h = jnp.maximum(d[:, None] * w1_ref[0, :][None, :] + b1_ref[0, :][None, :], 0.0)
    mlp = jnp.dot(h, w2_ref[...].T, preferred_element_type=jnp.float32)
    mlp = jnp.maximum(mlp + b2_ref[0, :][None, :], 0.0)
    out_ref[...] = jnp.concatenate([one_hot, mlp], axis=1)


def _eac_pallas(edge_attr, W1, b1, W2, b2):
    grid = (E // E_BLK,)
    return pl.pallas_call(
        _eac_body,
        grid=grid,
        in_specs=[
            pl.BlockSpec((E_BLK, 1), lambda i: (i, i * 0)),
            pl.BlockSpec((1, EH), lambda i: (i * 0, i * 0)),
            pl.BlockSpec((1, EH), lambda i: (i * 0, i * 0)),
            pl.BlockSpec((EO, EH), lambda i: (i * 0, i * 0)),
            pl.BlockSpec((1, EO), lambda i: (i * 0, i * 0)),
        ],
        out_specs=pl.BlockSpec((E_BLK, H), lambda i: (i, i * 0)),
        out_shape=jax.ShapeDtypeStruct((E, H), jnp.float32),
    )(edge_attr, W1.T.reshape(1, EH), b1.reshape(1, EH), W2, b2.reshape(1, EO))


def _final_body(xs_ref, sf_ref, g1_ref, g2_ref, bias_ref, out_ref):
    xs = xs_ref[...]
    sf = sf_ref[...]
    out0 = jnp.dot(xs, g1_ref[...].T, preferred_element_type=jnp.float32)
    out0 = out0 + jnp.dot(sf, g2_ref[...].T, preferred_element_type=jnp.float32)
    out0 = jnp.maximum(out0 + bias_ref[0, :][None, :], 0.0)
    out_ref[...] = jnp.stack([out0, sf], axis=1)


def _final_pallas(xs, sum_features, gamma1, gamma2, bias):
    grid = (N // N_BLK,)
    return pl.pallas_call(
        _final_body,
        grid=grid,
        in_specs=[
            pl.BlockSpec((N_BLK, H), lambda i: (i, i * 0)),
            pl.BlockSpec((N_BLK, H), lambda i: (i, i * 0)),
            pl.BlockSpec((H, H), lambda i: (i * 0, i * 0)),
            pl.BlockSpec((H, H), lambda i: (i * 0, i * 0)),
            pl.BlockSpec((1, H), lambda i: (i * 0, i * 0)),
        ],
        out_specs=pl.BlockSpec((N_BLK, 2, H), lambda i: (i, i * 0, i * 0)),
        out_shape=jax.ShapeDtypeStruct((N, 2, H), jnp.float32),
    )(xs, sum_features, gamma1, gamma2, bias.reshape(1, H))


def kernel(x, edge_index, edge_attr, a, b, gamma1, gamma2, bias, W1, b1, W2, b2):
    src = edge_index[0]
    dst = edge_index[1]

    eac = _eac_pallas(edge_attr, W1, b1, W2, b2)

    keys = src * N + dst
    order = jnp.argsort(keys)
    sorted_keys = keys[order]
    pos_self = jnp.searchsorted(sorted_keys, keys)
    first_idx = order[pos_self]
    keep = (first_idx == jnp.arange(E)).astype(jnp.float32)
    rev_keys = dst * N + src
    pos_rev = jnp.clip(jnp.searchsorted(sorted_keys, rev_keys), 0, E - 1)
    rev_exists = sorted_keys[pos_rev] == rev_keys
    rev_idx = order[pos_rev]
    use_rev = (src > dst) & rev_exists
    sel_idx = jnp.where(use_rev, rev_idx, jnp.arange(E))
    denom_node = jnp.where(use_rev, dst, src)

    sum_w = jax.ops.segment_sum(eac, src, num_segments=N)

    denom = sum_w[denom_node]
    eac_sel = eac[sel_idx]
    w_used = jnp.where(denom != 0, eac_sel / jnp.where(denom != 0, denom, 1.0),
                       jnp.float32(0.01))

    xs = x[:, 0, :]
    rho = jnp.abs(a * xs[src] - (1.0 - a) * xs[dst]) ** b
    contrib = rho * w_used * keep[:, None]
    sum_features = jax.ops.segment_sum(contrib, src, num_segments=N)

    return _final_pallas(xs, sum_features, gamma1, gamma2, bias)


# speedup vs baseline: 3.8862x; 3.6745x over previous
import functools

import jax
import jax.numpy as jnp
from jax import lax
from jax.experimental import pallas as pl
from jax.experimental.pallas import tpu as pltpu

N = 100000
E = 1600000
H = 20
EH = 32
EO = 10

E_BLK = 12800
N_BLK = 1000


def _eac_body(attr_ref, w1_ref, b1_ref, w2_ref, b2_ref, out_ref):
    d = attr_ref[:, 0]
    v = d / jnp.float32(0.1)
    bin_idx = jnp.clip(v.astype(jnp.int32), 0, 9)
    cols = lax.broadcasted_iota(jnp.int32, (d.shape[0], 10), 1)
    one_hot = (cols == bin_idx[:, None]).astype(jnp.float32)
    h = jnp.maximum(d[:, None] * w1_ref[0, :][None, :] + b1_ref[0, :][None, :], 0.0)
    mlp = jnp.dot(h, w2_ref[...].T, preferred_element_type=jnp.float32)
    mlp = jnp.maximum(mlp + b2_ref[0, :][None, :], 0.0)
    out_ref[...] = jnp.concatenate([one_hot, mlp], axis=1)


def _eac_pallas(edge_attr, W1, b1, W2, b2):
    grid = (E // E_BLK,)
    return pl.pallas_call(
        _eac_body,
        grid=grid,
        in_specs=[
            pl.BlockSpec((E_BLK, 1), lambda i: (i, i * 0)),
            pl.BlockSpec((1, EH), lambda i: (i * 0, i * 0)),
            pl.BlockSpec((1, EH), lambda i: (i * 0, i * 0)),
            pl.BlockSpec((EO, EH), lambda i: (i * 0, i * 0)),
            pl.BlockSpec((1, EO), lambda i: (i * 0, i * 0)),
        ],
        out_specs=pl.BlockSpec((E_BLK, H), lambda i: (i, i * 0)),
        out_shape=jax.ShapeDtypeStruct((E, H), jnp.float32),
    )(edge_attr, W1.T.reshape(1, EH), b1.reshape(1, EH), W2, b2.reshape(1, EO))


def _final_body(xs_ref, sf_ref, g1_ref, g2_ref, bias_ref, out_ref):
    xs = xs_ref[...]
    sf = sf_ref[...]
    out0 = jnp.dot(xs, g1_ref[...].T, preferred_element_type=jnp.float32)
    out0 = out0 + jnp.dot(sf, g2_ref[...].T, preferred_element_type=jnp.float32)
    out0 = jnp.maximum(out0 + bias_ref[0, :][None, :], 0.0)
    out_ref[...] = jnp.stack([out0, sf], axis=1)


def _final_pallas(xs, sum_features, gamma1, gamma2, bias):
    grid = (N // N_BLK,)
    return pl.pallas_call(
        _final_body,
        grid=grid,
        in_specs=[
            pl.BlockSpec((N_BLK, H), lambda i: (i, i * 0)),
            pl.BlockSpec((N_BLK, H), lambda i: (i, i * 0)),
            pl.BlockSpec((H, H), lambda i: (i * 0, i * 0)),
            pl.BlockSpec((H, H), lambda i: (i * 0, i * 0)),
            pl.BlockSpec((1, H), lambda i: (i * 0, i * 0)),
        ],
        out_specs=pl.BlockSpec((N_BLK, 2, H), lambda i: (i, i * 0, i * 0)),
        out_shape=jax.ShapeDtypeStruct((N, 2, H), jnp.float32),
    )(xs, sum_features, gamma1, gamma2, bias.reshape(1, H))


def kernel(x, edge_index, edge_attr, a, b, gamma1, gamma2, bias, W1, b1, W2, b2):
    src = edge_index[0]
    dst = edge_index[1]

    eac = _eac_pallas(edge_attr, W1, b1, W2, b2)

    s32 = src.astype(jnp.int32)
    d32 = dst.astype(jnp.int32)
    mn = jnp.minimum(s32, d32)
    mx = jnp.maximum(s32, d32)
    tag = (s32 > d32).astype(jnp.int32)
    mxt = mx * 2 + tag
    iota = jnp.arange(E, dtype=jnp.int32)
    mxt_p, mn_p, pay_p = lax.sort((mxt, mn, iota), num_keys=1, is_stable=True)
    mn_s, mxt_s, pay_s = lax.sort((mn_p, mxt_p, pay_p), num_keys=1, is_stable=True)
    prev_mn = jnp.concatenate([jnp.full((1,), -1, jnp.int32), mn_s[:-1]])
    prev_mxt = jnp.concatenate([jnp.full((1,), -1, jnp.int32), mxt_s[:-1]])
    new_run = (mn_s != prev_mn) | ((mxt_s >> 1) != (prev_mxt >> 1))
    new_subrun = (mn_s != prev_mn) | (mxt_s != prev_mxt)
    run_start = lax.cummax(jnp.where(new_run, iota, jnp.int32(-1)))
    subrun_start = lax.cummax(jnp.where(new_subrun, iota, jnp.int32(-1)))
    keep_sorted = (iota == subrun_start)
    tag_s = mxt_s & 1
    packed_arr = pay_s | (tag_s << 21)
    g = packed_arr[run_start]
    rev_exists_s = (g >> 21) == 0
    rev_rep_s = g & 0x1FFFFF
    is_rev_q = tag_s == 1
    out_packed = (keep_sorted.astype(jnp.int32)
                  | jnp.where(is_rev_q & rev_exists_s, jnp.int32(2), jnp.int32(0))
                  | jnp.where(is_rev_q & rev_exists_s, rev_rep_s << 2, jnp.int32(0)))
    res = jnp.zeros((E,), jnp.int32).at[pay_s].add(out_packed)
    keep = (res & 1).astype(jnp.float32)
    use_rev = ((res >> 1) & 1) == 1
    sel_idx = jnp.where(use_rev, res >> 2, iota)
    denom_node = jnp.where(use_rev, d32, s32)

    sum_w = jax.ops.segment_sum(eac, src, num_segments=N)

    denom = sum_w[denom_node]
    eac_sel = eac[sel_idx]
    w_used = jnp.where(denom != 0, eac_sel / jnp.where(denom != 0, denom, 1.0),
                       jnp.float32(0.01))

    xs = x[:, 0, :]
    rho = jnp.abs(a * xs[src] - (1.0 - a) * xs[dst]) ** b
    contrib = rho * w_used * keep[:, None]
    sum_features = jax.ops.segment_sum(contrib, src, num_segments=N)

    return _final_pallas(xs, sum_features, gamma1, gamma2, bias)
